# Initial kernel scaffold; baseline (speedup 1.0000x reference)
import jax
import jax.numpy as jnp
from jax.experimental import pallas as pl


def kernel(x, edge_index, W1, b1, W2, b2, W3, b3, W4, b4):
    raise NotImplementedError("write your pallas kernel here")


# speedup vs baseline: 9.4943x; 9.4943x over previous
import functools

import jax
import jax.numpy as jnp
from jax import lax
from jax.experimental import pallas as pl
from jax.experimental.pallas import tpu as pltpu
from jax.experimental.pallas import tpu_sc as plsc

N_NODES = 10000
N_EDGES = 320000
D_IN = 128
D_HID = 128
D_OUT = 64

NC = 2
NS = 16
NW = NC * NS
EDGES_PER_W = N_EDGES // NW
CHUNK = 80
N_CHUNKS = EDGES_PER_W // CHUNK
STRIPE_A = 632
STRIPE_B = N_NODES - 15 * STRIPE_A


def _make_propagate(d: int):
  mesh = plsc.VectorSubcoreMesh(core_axis_name="c", subcore_axis_name="s")

  def body(table_hbm, src_hbm, dst_hbm, p_hbm, idxs_v, idxd_v, rows_v, acc, sem):
    c = lax.axis_index("c")
    s = lax.axis_index("s")
    w = c * NS + s
    r0 = s * STRIPE_A

    @pl.when(s < NS - 1)
    def _():
      pltpu.sync_copy(table_hbm.at[pl.ds(r0, STRIPE_A)],
                      acc.at[pl.ds(r0, STRIPE_A)])

    @pl.when(s == NS - 1)
    def _():
      pltpu.sync_copy(table_hbm.at[pl.ds(15 * STRIPE_A, STRIPE_B)],
                      acc.at[pl.ds(15 * STRIPE_A, STRIPE_B)])

    plsc.subcore_barrier()

    def step(i, carry):
      base = w * EDGES_PER_W + i * CHUNK
      pltpu.sync_copy(src_hbm.at[pl.ds(base, CHUNK)], idxs_v)
      pltpu.sync_copy(dst_hbm.at[pl.ds(base, CHUNK)], idxd_v)
      pltpu.async_copy(table_hbm.at[idxs_v], rows_v, sem).wait()
      pltpu.sync_copy(rows_v, acc.at[idxd_v], add=True)
      return carry

    lax.fori_loop(0, N_CHUNKS, step, 0)
    plsc.subcore_barrier()

    @pl.when(s < NS - 1)
    def _():
      pltpu.sync_copy(acc.at[pl.ds(r0, STRIPE_A)],
                      p_hbm.at[c, pl.ds(r0, STRIPE_A)])

    @pl.when(s == NS - 1)
    def _():
      pltpu.sync_copy(acc.at[pl.ds(15 * STRIPE_A, STRIPE_B)],
                      p_hbm.at[c, pl.ds(15 * STRIPE_A, STRIPE_B)])

  return pl.kernel(
      body,
      out_type=jax.ShapeDtypeStruct((NC, N_NODES, d), jnp.float32),
      mesh=mesh,
      compiler_params=pltpu.CompilerParams(use_tc_tiling_on_sc=False),
      scratch_types=[
          pltpu.VMEM((CHUNK,), jnp.int32),
          pltpu.VMEM((CHUNK,), jnp.int32),
          pltpu.VMEM((CHUNK, d), jnp.float32),
          pltpu.VMEM_SHARED((N_NODES, d), jnp.float32),
          pltpu.SemaphoreType.DMA,
      ],
  )


_propagate_128 = _make_propagate(D_HID)
_propagate_64 = _make_propagate(D_OUT)
_propagate_16 = _make_propagate(16)


BN = 1000
GRID = N_NODES // BN


def _dinv_from_degp(degp_blk):
  deg = degp_blk[0, :, 0:1] + degp_blk[1, :, 0:1] - 1.0
  return lax.rsqrt(jnp.maximum(deg, 1e-12))


def _mm_first_body(x_ref, w_ref, degp_ref, out_ref):
  dinv = _dinv_from_degp(degp_ref[...])
  out_ref[...] = dinv * jnp.dot(x_ref[...], w_ref[...],
                                preferred_element_type=jnp.float32)


def _mm_mid_body(p_ref, hp_ref, b_ref, w_ref, degp_ref, out_ref):
  dinv = _dinv_from_degp(degp_ref[...])
  pp = p_ref[...]
  z = dinv * (pp[0] + pp[1] - hp_ref[...]) + b_ref[...]
  a = jnp.maximum(z, 0.0)
  out_ref[...] = dinv * jnp.dot(a, w_ref[...],
                                preferred_element_type=jnp.float32)


def _soft_body(p_ref, hp_ref, b_ref, degp_ref, out_ref):
  dinv = _dinv_from_degp(degp_ref[...])
  pp = p_ref[...]
  z = dinv * (pp[0] + pp[1] - hp_ref[...]) + b_ref[...]
  z = z - jnp.max(z, axis=1, keepdims=True)
  ez = jnp.exp(z)
  out_ref[...] = ez / jnp.sum(ez, axis=1, keepdims=True)


def _row_blk(d):
  return pl.BlockSpec((BN, d), lambda i: (i, 0))


def _p_blk(d):
  return pl.BlockSpec((NC, BN, d), lambda i: (0, i, 0))


_DEGP_BLK = pl.BlockSpec((NC, BN, 16), lambda i: (0, i, 0))


def _full_blk(a, b):
  return pl.BlockSpec((a, b), lambda i: (0, 0))


def _mm_first(x, w, degp):
  return pl.pallas_call(
      _mm_first_body,
      grid=(GRID,),
      in_specs=[_row_blk(D_IN), _full_blk(D_IN, D_HID), _DEGP_BLK],
      out_specs=_row_blk(D_HID),
      out_shape=jax.ShapeDtypeStruct((N_NODES, D_HID), jnp.float32),
  )(x, w, degp)


def _mm_mid(p, hp, b, w, degp, d_out):
  return pl.pallas_call(
      _mm_mid_body,
      grid=(GRID,),
      in_specs=[_p_blk(D_HID), _row_blk(D_HID), _full_blk(1, D_HID),
                _full_blk(D_HID, d_out), _DEGP_BLK],
      out_specs=_row_blk(d_out),
      out_shape=jax.ShapeDtypeStruct((N_NODES, d_out), jnp.float32),
  )(p, hp, b, w, degp)


def _softmax_out(p, hp, b, degp):
  return pl.pallas_call(
      _soft_body,
      grid=(GRID,),
      in_specs=[_p_blk(D_OUT), _row_blk(D_OUT), _full_blk(1, D_OUT), _DEGP_BLK],
      out_specs=_row_blk(D_OUT),
      out_shape=jax.ShapeDtypeStruct((N_NODES, D_OUT), jnp.float32),
  )(p, hp, b, degp)


def kernel(x, edge_index, W1, b1, W2, b2, W3, b3, W4, b4):
  src = edge_index[0].astype(jnp.int32)
  dst = edge_index[1].astype(jnp.int32)
  x = x.astype(jnp.float32)

  ones_tab = jnp.ones((N_NODES, 16), jnp.float32)
  degp = _propagate_16(ones_tab, dst, dst)

  h1p = _mm_first(x, W1, degp)
  p1 = _propagate_128(h1p, src, dst)
  h2p = _mm_mid(p1, h1p, b1.reshape(1, -1), W2, degp, D_HID)
  p2 = _propagate_128(h2p, src, dst)
  h3p = _mm_mid(p2, h2p, b2.reshape(1, -1), W3, degp, D_HID)
  p3 = _propagate_128(h3p, src, dst)
  h4p = _mm_mid(p3, h3p, b3.reshape(1, -1), W4, degp, D_OUT)
  p4 = _propagate_64(h4p, src, dst)
  return _softmax_out(p4, h4p, b4.reshape(1, -1), degp)


# speedup vs baseline: 29.9256x; 3.1520x over previous
import jax
import jax.numpy as jnp
from jax import lax
from jax.experimental import pallas as pl
from jax.experimental.pallas import tpu as pltpu
from jax.experimental.pallas import tpu_sc as plsc

N_NODES = 10000
N_EDGES = 320000
D_IN = 128
D_HID = 128
D_OUT = 64

NC = 2
NS = 16
NW = NC * NS
EDGES_PER_W = N_EDGES // NW
CHUNK = 40
N_CHUNKS = EDGES_PER_W // CHUNK
RING = 5
N_GROUPS = N_CHUNKS // RING
STRIPE_A = 632
STRIPE_B = N_NODES - 15 * STRIPE_A

_SC_PARAMS = pltpu.CompilerParams(use_tc_tiling_on_sc=False)


def _stripe_copy(s, src_at, dst_at):
  r0 = s * STRIPE_A

  @pl.when(s < NS - 1)
  def _():
    pltpu.sync_copy(src_at(pl.ds(r0, STRIPE_A)), dst_at(pl.ds(r0, STRIPE_A)))

  @pl.when(s == NS - 1)
  def _():
    pltpu.sync_copy(src_at(pl.ds(15 * STRIPE_A, STRIPE_B)),
                    dst_at(pl.ds(15 * STRIPE_A, STRIPE_B)))


def _make_propagate(d: int):
  mesh = plsc.VectorSubcoreMesh(core_axis_name="c", subcore_axis_name="s")

  def body(table_hbm, src_hbm, dst_hbm, p_hbm, srcv, dstv, rows, acc,
           gsems, ssems):
    c = lax.axis_index("c")
    s = lax.axis_index("s")
    w = c * NS + s

    pltpu.sync_copy(src_hbm.at[w], srcv)
    pltpu.sync_copy(dst_hbm.at[w], dstv)
    _stripe_copy(s, lambda sl: table_hbm.at[sl], lambda sl: acc.at[sl])
    plsc.subcore_barrier()

    def fire_gather(i, j):
      return pltpu.async_copy(table_hbm.at[srcv.at[i]], rows[j], gsems[j])

    def fire_scatter(i, j):
      return pltpu.async_copy(rows[j], acc.at[dstv.at[i]], ssems[j], add=True)

    for j in range(RING):
      fire_gather(j, j)

    def group(n, carry):
      for j in range(RING):
        i = n * RING + j
        pltpu.make_async_copy(table_hbm.at[srcv.at[i]], rows[j],
                              gsems[j]).wait()
        fire_scatter(i, j)

        @pl.when(i + RING < N_CHUNKS)
        def _():
          pltpu.make_async_copy(rows[j], acc.at[dstv.at[i]], ssems[j]).wait()
          fire_gather(i + RING, j)
      return carry

    lax.fori_loop(0, N_GROUPS, group, 0)
    for j in range(RING):
      i = N_CHUNKS - RING + j
      pltpu.make_async_copy(rows[j], acc.at[dstv.at[i]], ssems[j]).wait()

    plsc.subcore_barrier()
    _stripe_copy(s, lambda sl: acc.at[sl], lambda sl: p_hbm.at[c, sl])

  return pl.kernel(
      body,
      out_type=jax.ShapeDtypeStruct((NC, N_NODES, d), jnp.float32),
      mesh=mesh,
      compiler_params=_SC_PARAMS,
      scratch_types=[
          pltpu.VMEM((N_CHUNKS, CHUNK), jnp.int32),
          pltpu.VMEM((N_CHUNKS, CHUNK), jnp.int32),
          [pltpu.VMEM((CHUNK, d), jnp.float32) for _ in range(RING)],
          pltpu.VMEM_SHARED((N_NODES, d), jnp.float32),
          [pltpu.SemaphoreType.DMA for _ in range(RING)],
          [pltpu.SemaphoreType.DMA for _ in range(RING)],
      ],
  )


_propagate_128 = _make_propagate(D_HID)
_propagate_64 = _make_propagate(D_OUT)


def _make_degree():
  mesh = plsc.VectorSubcoreMesh(core_axis_name="c", subcore_axis_name="s")
  DD = 16

  def body(ones_hbm, dst_hbm, p_hbm, dstv, ones_v, acc, ssems):
    c = lax.axis_index("c")
    s = lax.axis_index("s")
    w = c * NS + s

    pltpu.sync_copy(dst_hbm.at[w], dstv)
    pltpu.sync_copy(ones_hbm.at[pl.ds(0, CHUNK)], ones_v)
    _stripe_copy(s, lambda sl: ones_hbm.at[sl], lambda sl: acc.at[sl])
    plsc.subcore_barrier()

    def fire(i, j):
      return pltpu.async_copy(ones_v, acc.at[dstv.at[i]], ssems[j], add=True)

    def group(n, carry):
      for j in range(RING):
        i = n * RING + j

        @pl.when(n > 0)
        def _():
          pltpu.make_async_copy(ones_v, acc.at[dstv.at[i]], ssems[j]).wait()

        fire(i, j)
      return carry

    lax.fori_loop(0, N_GROUPS, group, 0)
    for j in range(RING):
      pltpu.make_async_copy(ones_v, acc.at[dstv.at[0]], ssems[j]).wait()

    plsc.subcore_barrier()
    _stripe_copy(s, lambda sl: acc.at[sl], lambda sl: p_hbm.at[c, sl])

  return pl.kernel(
      body,
      out_type=jax.ShapeDtypeStruct((NC, N_NODES, DD), jnp.float32),
      mesh=mesh,
      compiler_params=_SC_PARAMS,
      scratch_types=[
          pltpu.VMEM((N_CHUNKS, CHUNK), jnp.int32),
          pltpu.VMEM((CHUNK, DD), jnp.float32),
          pltpu.VMEM_SHARED((N_NODES, DD), jnp.float32),
          [pltpu.SemaphoreType.DMA for _ in range(RING)],
      ],
  )


_degree = _make_degree()


BN = 1000
GRID = N_NODES // BN


def _dinv_from_degp(degp_blk):
  deg = degp_blk[0, :, 0:1] + degp_blk[1, :, 0:1] - 1.0
  return lax.rsqrt(jnp.maximum(deg, 1e-12))


def _mm_first_body(x_ref, w_ref, degp_ref, out_ref):
  dinv = _dinv_from_degp(degp_ref[...])
  out_ref[...] = dinv * jnp.dot(x_ref[...], w_ref[...],
                                preferred_element_type=jnp.float32)


def _mm_mid_body(p_ref, hp_ref, b_ref, w_ref, degp_ref, out_ref):
  dinv = _dinv_from_degp(degp_ref[...])
  pp = p_ref[...]
  z = dinv * (pp[0] + pp[1] - hp_ref[...]) + b_ref[...]
  a = jnp.maximum(z, 0.0)
  out_ref[...] = dinv * jnp.dot(a, w_ref[...],
                                preferred_element_type=jnp.float32)


def _soft_body(p_ref, hp_ref, b_ref, degp_ref, out_ref):
  dinv = _dinv_from_degp(degp_ref[...])
  pp = p_ref[...]
  z = dinv * (pp[0] + pp[1] - hp_ref[...]) + b_ref[...]
  z = z - jnp.max(z, axis=1, keepdims=True)
  ez = jnp.exp(z)
  out_ref[...] = ez / jnp.sum(ez, axis=1, keepdims=True)


def _row_blk(d):
  return pl.BlockSpec((BN, d), lambda i: (i, 0))


def _p_blk(d):
  return pl.BlockSpec((NC, BN, d), lambda i: (0, i, 0))


_DEGP_BLK = pl.BlockSpec((NC, BN, 16), lambda i: (0, i, 0))


def _full_blk(a, b):
  return pl.BlockSpec((a, b), lambda i: (0, 0))


def _mm_first(x, w, degp):
  return pl.pallas_call(
      _mm_first_body,
      grid=(GRID,),
      in_specs=[_row_blk(D_IN), _full_blk(D_IN, D_HID), _DEGP_BLK],
      out_specs=_row_blk(D_HID),
      out_shape=jax.ShapeDtypeStruct((N_NODES, D_HID), jnp.float32),
  )(x, w, degp)


def _mm_mid(p, hp, b, w, degp, d_out):
  return pl.pallas_call(
      _mm_mid_body,
      grid=(GRID,),
      in_specs=[_p_blk(D_HID), _row_blk(D_HID), _full_blk(1, D_HID),
                _full_blk(D_HID, d_out), _DEGP_BLK],
      out_specs=_row_blk(d_out),
      out_shape=jax.ShapeDtypeStruct((N_NODES, d_out), jnp.float32),
  )(p, hp, b, w, degp)


def _softmax_out(p, hp, b, degp):
  return pl.pallas_call(
      _soft_body,
      grid=(GRID,),
      in_specs=[_p_blk(D_OUT), _row_blk(D_OUT), _full_blk(1, D_OUT), _DEGP_BLK],
      out_specs=_row_blk(D_OUT),
      out_shape=jax.ShapeDtypeStruct((N_NODES, D_OUT), jnp.float32),
  )(p, hp, b, degp)


def kernel(x, edge_index, W1, b1, W2, b2, W3, b3, W4, b4):
  src = edge_index[0].astype(jnp.int32).reshape(NW, N_CHUNKS, CHUNK)
  dst = edge_index[1].astype(jnp.int32).reshape(NW, N_CHUNKS, CHUNK)
  x = x.astype(jnp.float32)

  ones_tab = jnp.ones((N_NODES, 16), jnp.float32)
  degp = _degree(ones_tab, dst)

  h1p = _mm_first(x, W1, degp)
  p1 = _propagate_128(h1p, src, dst)
  h2p = _mm_mid(p1, h1p, b1.reshape(1, -1), W2, degp, D_HID)
  p2 = _propagate_128(h2p, src, dst)
  h3p = _mm_mid(p2, h2p, b2.reshape(1, -1), W3, degp, D_HID)
  p3 = _propagate_128(h3p, src, dst)
  h4p = _mm_mid(p3, h3p, b3.reshape(1, -1), W4, degp, D_OUT)
  p4 = _propagate_64(h4p, src, dst)
  return _softmax_out(p4, h4p, b4.reshape(1, -1), degp)


# speedup vs baseline: 33.7630x; 1.1282x over previous
import jax
import jax.numpy as jnp
from jax import lax
from jax.experimental import pallas as pl
from jax.experimental.pallas import tpu as pltpu
from jax.experimental.pallas import tpu_sc as plsc

N_NODES = 10000
N_EDGES = 320000
D_IN = 128
D_HID = 128
D_OUT = 64

NC = 2
NS = 16
NW = NC * NS
EDGES_PER_W = N_EDGES // NW
CHUNK = 40
N_CHUNKS = EDGES_PER_W // CHUNK
RING = 5
N_GROUPS = N_CHUNKS // RING
STRIPE_A = 632
STRIPE_B = N_NODES - 15 * STRIPE_A

_SC_PARAMS = pltpu.CompilerParams(use_tc_tiling_on_sc=False)


def _stripe_copy(s, src_at, dst_at):
  r0 = s * STRIPE_A

  @pl.when(s < NS - 1)
  def _():
    pltpu.sync_copy(src_at(pl.ds(r0, STRIPE_A)), dst_at(pl.ds(r0, STRIPE_A)))

  @pl.when(s == NS - 1)
  def _():
    pltpu.sync_copy(src_at(pl.ds(15 * STRIPE_A, STRIPE_B)),
                    dst_at(pl.ds(15 * STRIPE_A, STRIPE_B)))


def _make_propagate(d: int, dtype=jnp.bfloat16, chunk=CHUNK):
  mesh = plsc.VectorSubcoreMesh(core_axis_name="c", subcore_axis_name="s")
  n_chunks = EDGES_PER_W // chunk
  n_groups = n_chunks // RING

  def body(table_hbm, src_hbm, dst_hbm, p_hbm, srcv, dstv, rows, acc,
           gsems, ssems):
    c = lax.axis_index("c")
    s = lax.axis_index("s")
    w = c * NS + s

    pltpu.sync_copy(src_hbm.at[w], srcv)
    pltpu.sync_copy(dst_hbm.at[w], dstv)
    _stripe_copy(s, lambda sl: table_hbm.at[sl], lambda sl: acc.at[sl])
    plsc.subcore_barrier()

    def fire_gather(i, j):
      return pltpu.async_copy(table_hbm.at[srcv.at[i]], rows[j], gsems[j])

    def fire_scatter(i, j):
      return pltpu.async_copy(rows[j], acc.at[dstv.at[i]], ssems[j], add=True)

    for j in range(RING):
      fire_gather(j, j)

    def group(n, carry):
      for j in range(RING):
        i = n * RING + j
        pltpu.make_async_copy(table_hbm.at[srcv.at[i]], rows[j],
                              gsems[j]).wait()
        fire_scatter(i, j)

        @pl.when(i + RING < n_chunks)
        def _():
          pltpu.make_async_copy(rows[j], acc.at[dstv.at[i]], ssems[j]).wait()
          fire_gather(i + RING, j)
      return carry

    lax.fori_loop(0, n_groups, group, 0)
    for j in range(RING):
      i = n_chunks - RING + j
      pltpu.make_async_copy(rows[j], acc.at[dstv.at[i]], ssems[j]).wait()

    plsc.subcore_barrier()
    _stripe_copy(s, lambda sl: acc.at[sl], lambda sl: p_hbm.at[c, sl])

  return pl.kernel(
      body,
      out_type=jax.ShapeDtypeStruct((NC, N_NODES, d), dtype),
      mesh=mesh,
      compiler_params=_SC_PARAMS,
      scratch_types=[
          pltpu.VMEM((n_chunks, chunk), jnp.int32),
          pltpu.VMEM((n_chunks, chunk), jnp.int32),
          [pltpu.VMEM((chunk, d), dtype) for _ in range(RING)],
          pltpu.VMEM_SHARED((N_NODES, d), dtype),
          [pltpu.SemaphoreType.DMA for _ in range(RING)],
          [pltpu.SemaphoreType.DMA for _ in range(RING)],
      ],
  )


_propagate_128 = _make_propagate(D_HID, jnp.bfloat16, 80)
_propagate_64 = _make_propagate(D_OUT, jnp.bfloat16, 80)


def _make_degree():
  mesh = plsc.VectorSubcoreMesh(core_axis_name="c", subcore_axis_name="s")
  DD = 16

  def body(ones_hbm, dst_hbm, p_hbm, dstv, ones_v, acc, ssems):
    c = lax.axis_index("c")
    s = lax.axis_index("s")
    w = c * NS + s

    pltpu.sync_copy(dst_hbm.at[w], dstv)
    pltpu.sync_copy(ones_hbm.at[pl.ds(0, CHUNK)], ones_v)
    _stripe_copy(s, lambda sl: ones_hbm.at[sl], lambda sl: acc.at[sl])
    plsc.subcore_barrier()

    def fire(i, j):
      return pltpu.async_copy(ones_v, acc.at[dstv.at[i]], ssems[j], add=True)

    def group(n, carry):
      for j in range(RING):
        i = n * RING + j

        @pl.when(n > 0)
        def _():
          pltpu.make_async_copy(ones_v, acc.at[dstv.at[i]], ssems[j]).wait()

        fire(i, j)
      return carry

    lax.fori_loop(0, N_GROUPS, group, 0)
    for j in range(RING):
      pltpu.make_async_copy(ones_v, acc.at[dstv.at[0]], ssems[j]).wait()

    plsc.subcore_barrier()
    _stripe_copy(s, lambda sl: acc.at[sl], lambda sl: p_hbm.at[c, sl])

  return pl.kernel(
      body,
      out_type=jax.ShapeDtypeStruct((NC, N_NODES, DD), jnp.float32),
      mesh=mesh,
      compiler_params=_SC_PARAMS,
      scratch_types=[
          pltpu.VMEM((N_CHUNKS, CHUNK), jnp.int32),
          pltpu.VMEM((CHUNK, DD), jnp.float32),
          pltpu.VMEM_SHARED((N_NODES, DD), jnp.float32),
          [pltpu.SemaphoreType.DMA for _ in range(RING)],
      ],
  )


_degree = _make_degree()


BN = 1000
GRID = N_NODES // BN


def _dinv_from_degp(degp_blk):
  deg = degp_blk[0, :, 0:1] + degp_blk[1, :, 0:1] - 1.0
  return lax.rsqrt(jnp.maximum(deg, 1e-12))


def _mm_first_body(x_ref, w_ref, degp_ref, out_ref):
  dinv = _dinv_from_degp(degp_ref[...])
  out_ref[...] = (dinv * jnp.dot(x_ref[...], w_ref[...],
                                 preferred_element_type=jnp.float32)
                  ).astype(out_ref.dtype)


def _mm_mid_body(p_ref, hp_ref, b_ref, w_ref, degp_ref, out_ref):
  dinv = _dinv_from_degp(degp_ref[...])
  pp = p_ref[...].astype(jnp.float32)
  z = dinv * (pp[0] + pp[1] - hp_ref[...].astype(jnp.float32)) + b_ref[...]
  a = jnp.maximum(z, 0.0)
  out_ref[...] = (dinv * jnp.dot(a, w_ref[...],
                                 preferred_element_type=jnp.float32)
                  ).astype(out_ref.dtype)


def _soft_body(p_ref, hp_ref, b_ref, degp_ref, out_ref):
  dinv = _dinv_from_degp(degp_ref[...])
  pp = p_ref[...].astype(jnp.float32)
  z = dinv * (pp[0] + pp[1] - hp_ref[...].astype(jnp.float32)) + b_ref[...]
  z = z - jnp.max(z, axis=1, keepdims=True)
  ez = jnp.exp(z)
  out_ref[...] = ez / jnp.sum(ez, axis=1, keepdims=True)


def _row_blk(d):
  return pl.BlockSpec((BN, d), lambda i: (i, 0))


def _p_blk(d):
  return pl.BlockSpec((NC, BN, d), lambda i: (0, i, 0))


_DEGP_BLK = pl.BlockSpec((NC, BN, 16), lambda i: (0, i, 0))


def _full_blk(a, b):
  return pl.BlockSpec((a, b), lambda i: (0, 0))


def _mm_first(x, w, degp):
  return pl.pallas_call(
      _mm_first_body,
      grid=(GRID,),
      in_specs=[_row_blk(D_IN), _full_blk(D_IN, D_HID), _DEGP_BLK],
      out_specs=_row_blk(D_HID),
      out_shape=jax.ShapeDtypeStruct((N_NODES, D_HID), jnp.bfloat16),
  )(x, w, degp)


def _mm_mid(p, hp, b, w, degp, d_out):
  return pl.pallas_call(
      _mm_mid_body,
      grid=(GRID,),
      in_specs=[_p_blk(D_HID), _row_blk(D_HID), _full_blk(1, D_HID),
                _full_blk(D_HID, d_out), _DEGP_BLK],
      out_specs=_row_blk(d_out),
      out_shape=jax.ShapeDtypeStruct((N_NODES, d_out), jnp.bfloat16),
  )(p, hp, b, w, degp)


def _softmax_out(p, hp, b, degp):
  return pl.pallas_call(
      _soft_body,
      grid=(GRID,),
      in_specs=[_p_blk(D_OUT), _row_blk(D_OUT), _full_blk(1, D_OUT), _DEGP_BLK],
      out_specs=_row_blk(D_OUT),
      out_shape=jax.ShapeDtypeStruct((N_NODES, D_OUT), jnp.float32),
  )(p, hp, b, degp)


def kernel(x, edge_index, W1, b1, W2, b2, W3, b3, W4, b4):
  src32 = edge_index[0].astype(jnp.int32)
  dst32 = edge_index[1].astype(jnp.int32)
  src = src32.reshape(NW, EDGES_PER_W // 80, 80)
  dst = dst32.reshape(NW, EDGES_PER_W // 80, 80)
  dst40 = dst32.reshape(NW, N_CHUNKS, CHUNK)
  x = x.astype(jnp.float32)

  ones_tab = jnp.ones((N_NODES, 16), jnp.float32)
  degp = _degree(ones_tab, dst40)

  h1p = _mm_first(x, W1, degp)
  p1 = _propagate_128(h1p, src, dst)
  h2p = _mm_mid(p1, h1p, b1.reshape(1, -1), W2, degp, D_HID)
  p2 = _propagate_128(h2p, src, dst)
  h3p = _mm_mid(p2, h2p, b2.reshape(1, -1), W3, degp, D_HID)
  p3 = _propagate_128(h3p, src, dst)
  h4p = _mm_mid(p3, h3p, b3.reshape(1, -1), W4, degp, D_OUT)
  p4 = _propagate_64(h4p, src, dst)
  return _softmax_out(p4, h4p, b4.reshape(1, -1), degp)
